# baseline (speedup 1.0000x reference)
import functools

import jax
import jax.numpy as jnp
from jax.experimental import pallas as pl
from jax.experimental.pallas import tpu as pltpu

T = 2048
H = 2048
NH = 16
DH = 128
E = 8
K = 2
DFF = 1024
EPS = 1e-6

BF16 = jnp.bfloat16
F32 = jnp.float32


def _ln_qkv_body(x_ref, g_ref, w_ref, o_ref):
    x = x_ref[...]
    v = jnp.mean(x * x, axis=1, keepdims=True)
    xn = (x * jax.lax.rsqrt(v + EPS) * g_ref[...]).astype(BF16)
    y = jax.lax.dot_general(xn, w_ref[...], (((1,), (0,)), ((), ())),
                            preferred_element_type=F32)
    o_ref[...] = y.astype(BF16)


def _ln_qkv(x, g, wqkv):
    BN = 512
    return pl.pallas_call(
        _ln_qkv_body,
        grid=(3 * H // BN,),
        in_specs=[
            pl.BlockSpec((T, H), lambda j: (0, 0)),
            pl.BlockSpec((1, H), lambda j: (0, 0)),
            pl.BlockSpec((H, BN), lambda j: (0, j)),
        ],
        out_specs=pl.BlockSpec((T, BN), lambda j: (0, j)),
        out_shape=jax.ShapeDtypeStruct((T, 3 * H), BF16),
    )(x, g.reshape(1, H), wqkv)


def _flash_body(q_ref, k_ref, v_ref, o_ref, m_ref, l_ref, acc_ref, *, bq, bk):
    i = pl.program_id(1)
    j = pl.program_id(2)
    scale = 1.0 / (DH ** 0.5)

    @pl.when(j == 0)
    def _():
        m_ref[...] = jnp.full_like(m_ref, -1e30)
        l_ref[...] = jnp.zeros_like(l_ref)
        acc_ref[...] = jnp.zeros_like(acc_ref)

    @pl.when(j <= i)
    def _():
        q = q_ref[...]
        k = k_ref[...]
        s = jax.lax.dot_general(q, k, (((1,), (1,)), ((), ())),
                                preferred_element_type=F32) * scale
        row = i * bq + jax.lax.broadcasted_iota(jnp.int32, (bq, bk), 0)
        col = j * bk + jax.lax.broadcasted_iota(jnp.int32, (bq, bk), 1)
        s = jnp.where(col > row, -1e9, s)
        m_prev = m_ref[:, :1]
        m_cur = jnp.max(s, axis=1, keepdims=True)
        m_new = jnp.maximum(m_prev, m_cur)
        p = jnp.exp(s - m_new)
        corr = jnp.exp(m_prev - m_new)
        l_ref[:, :1] = l_ref[:, :1] * corr + jnp.sum(p, axis=1, keepdims=True)
        acc_ref[...] = acc_ref[...] * corr + jax.lax.dot_general(
            p.astype(BF16), v_ref[...], (((1,), (0,)), ((), ())),
            preferred_element_type=F32)
        m_ref[:, :1] = m_new

    @pl.when(j == i)
    def _():
        o_ref[...] = (acc_ref[...] / l_ref[:, :1]).astype(BF16)


def _flash(qkv):
    BQ = 512
    BK = 512
    body = functools.partial(_flash_body, bq=BQ, bk=BK)
    return pl.pallas_call(
        body,
        grid=(NH, T // BQ, T // BK),
        in_specs=[
            pl.BlockSpec((BQ, DH), lambda h, i, j: (i, h)),
            pl.BlockSpec((BK, DH), lambda h, i, j: (j, NH + h)),
            pl.BlockSpec((BK, DH), lambda h, i, j: (j, 2 * NH + h)),
        ],
        out_specs=pl.BlockSpec((BQ, DH), lambda h, i, j: (i, h)),
        out_shape=jax.ShapeDtypeStruct((T, H), BF16),
        scratch_shapes=[
            pltpu.VMEM((BQ, 1), F32),
            pltpu.VMEM((BQ, 1), F32),
            pltpu.VMEM((BQ, DH), F32),
        ],
        compiler_params=pltpu.CompilerParams(
            dimension_semantics=("arbitrary", "arbitrary", "arbitrary")),
    )(qkv, qkv, qkv)


def _proj_ln2_body(o_ref, x_ref, wo_ref, g_ref, wr_ref,
                   h1_ref, x2n_ref, lg_ref):
    a = jax.lax.dot_general(o_ref[...], wo_ref[...], (((1,), (0,)), ((), ())),
                            preferred_element_type=F32)
    h1 = x_ref[...] + a
    h1_ref[...] = h1
    v = jnp.mean(h1 * h1, axis=1, keepdims=True)
    xn = h1 * jax.lax.rsqrt(v + EPS) * g_ref[...]
    x2n_ref[...] = xn.astype(BF16)
    lg_ref[...] = jax.lax.dot_general(xn, wr_ref[...], (((1,), (0,)), ((), ())),
                                      preferred_element_type=F32)


def _proj_ln2(o, x, wo, g2, wr):
    BM = 512
    return pl.pallas_call(
        _proj_ln2_body,
        grid=(T // BM,),
        in_specs=[
            pl.BlockSpec((BM, H), lambda i: (i, 0)),
            pl.BlockSpec((BM, H), lambda i: (i, 0)),
            pl.BlockSpec((H, H), lambda i: (0, 0)),
            pl.BlockSpec((1, H), lambda i: (0, 0)),
            pl.BlockSpec((H, E), lambda i: (0, 0)),
        ],
        out_specs=[
            pl.BlockSpec((BM, H), lambda i: (i, 0)),
            pl.BlockSpec((BM, H), lambda i: (i, 0)),
            pl.BlockSpec((BM, E), lambda i: (i, 0)),
        ],
        out_shape=[
            jax.ShapeDtypeStruct((T, H), F32),
            jax.ShapeDtypeStruct((T, H), BF16),
            jax.ShapeDtypeStruct((T, E), F32),
        ],
    )(o, x, wo, g2.reshape(1, H), wr)


def _route_body(lg_ref, comb_ref):
    l = lg_ref[...]
    col = jax.lax.broadcasted_iota(jnp.int32, (T, E), 1)
    m1 = jnp.max(l, axis=1, keepdims=True)
    a1 = jnp.min(jnp.where(l == m1, col, E), axis=1, keepdims=True)
    sel1 = col == a1
    l2 = jnp.where(sel1, -1e30, l)
    m2 = jnp.max(l2, axis=1, keepdims=True)
    a2 = jnp.min(jnp.where(l2 == m2, col, E), axis=1, keepdims=True)
    sel2 = col == a2
    w1 = jax.nn.sigmoid(m1 - m2)
    w2 = 1.0 - w1
    comb_ref[...] = jnp.where(sel1, w1, 0.0) + jnp.where(sel2, w2, 0.0)


def _route(logits):
    return pl.pallas_call(
        _route_body,
        out_shape=jax.ShapeDtypeStruct((T, E), F32),
    )(logits)


def _moe_body(x_ref, w1_ref, w2_ref, comb_ref, h1_ref, o_ref):
    e = pl.program_id(1)
    x = x_ref[...]
    gu = jax.lax.dot_general(x, w1_ref[0], (((1,), (1,)), ((), ())),
                             preferred_element_type=F32)
    g = gu[:, :DFF]
    u = gu[:, DFF:]
    act = (g * jax.nn.sigmoid(g) * u).astype(BF16)
    dn = jax.lax.dot_general(act, w2_ref[0], (((1,), (1,)), ((), ())),
                             preferred_element_type=F32)
    c = comb_ref[...]
    onehot = (jax.lax.broadcasted_iota(jnp.int32, c.shape, 1) == e)
    wgt = jnp.sum(jnp.where(onehot, c, 0.0), axis=1, keepdims=True)
    contrib = dn * wgt

    @pl.when(e == 0)
    def _():
        o_ref[...] = h1_ref[...] + contrib

    @pl.when(e > 0)
    def _():
        o_ref[...] = o_ref[...] + contrib


def _moe(x2n, w1, w2, comb, h1):
    BM = 512
    return pl.pallas_call(
        _moe_body,
        grid=(T // BM, E),
        in_specs=[
            pl.BlockSpec((BM, H), lambda i, e: (i, 0)),
            pl.BlockSpec((1, 2 * DFF, H), lambda i, e: (e, 0, 0)),
            pl.BlockSpec((1, H, DFF), lambda i, e: (e, 0, 0)),
            pl.BlockSpec((BM, E), lambda i, e: (i, 0)),
            pl.BlockSpec((BM, H), lambda i, e: (i, 0)),
        ],
        out_specs=pl.BlockSpec((BM, H), lambda i, e: (i, 0)),
        out_shape=jax.ShapeDtypeStruct((T, H), F32),
        compiler_params=pltpu.CompilerParams(
            dimension_semantics=("arbitrary", "arbitrary")),
    )(x2n, w1, w2, comb, h1)


def kernel(hidden_states, ln1_g, ln2_g, wq, wk, wv, wo, w_router, w1, w2):
    wqkv = jnp.concatenate([wq, wk, wv], axis=1).astype(BF16)
    qkv = _ln_qkv(hidden_states, ln1_g, wqkv)
    o = _flash(qkv)
    h1, x2n, logits = _proj_ln2(o, hidden_states, wo.astype(BF16), ln2_g,
                                w_router)
    comb = _route(logits)
    out = _moe(x2n, w1.astype(BF16), w2.astype(BF16), comb, h1)
    return out


# speedup vs baseline: 1.2275x; 1.2275x over previous
import functools

import jax
import jax.numpy as jnp
from jax.experimental import pallas as pl
from jax.experimental.pallas import tpu as pltpu

T = 2048
H = 2048
NH = 16
DH = 128
E = 8
K = 2
DFF = 1024
EPS = 1e-6

BF16 = jnp.bfloat16
F32 = jnp.float32


def _ln_qkv_body(x_ref, g_ref, w_ref, o_ref):
    x = x_ref[...]
    v = jnp.mean(x * x, axis=1, keepdims=True)
    xn = (x * jax.lax.rsqrt(v + EPS) * g_ref[...]).astype(BF16)
    y = jax.lax.dot_general(xn, w_ref[...], (((1,), (0,)), ((), ())),
                            preferred_element_type=F32)
    o_ref[...] = y.astype(BF16)


def _ln_qkv(x, g, wqkv):
    BN = 512
    return pl.pallas_call(
        _ln_qkv_body,
        grid=(3 * H // BN,),
        in_specs=[
            pl.BlockSpec((T, H), lambda j: (0, 0)),
            pl.BlockSpec((1, H), lambda j: (0, 0)),
            pl.BlockSpec((H, BN), lambda j: (0, j)),
        ],
        out_specs=pl.BlockSpec((T, BN), lambda j: (0, j)),
        out_shape=jax.ShapeDtypeStruct((T, 3 * H), BF16),
    )(x, g.reshape(1, H), wqkv)


def _flash_body(q_ref, k_ref, v_ref, o_ref, m_ref, l_ref, acc_ref, *, bq, bk):
    i = pl.program_id(1)
    j = pl.program_id(2)
    scale = 1.0 / (DH ** 0.5)

    @pl.when(j == 0)
    def _():
        m_ref[...] = jnp.full_like(m_ref, -1e30)
        l_ref[...] = jnp.zeros_like(l_ref)
        acc_ref[...] = jnp.zeros_like(acc_ref)

    @pl.when(j <= i)
    def _():
        q = q_ref[...]
        k = k_ref[...]
        s = jax.lax.dot_general(q, k, (((1,), (1,)), ((), ())),
                                preferred_element_type=F32) * scale
        row = i * bq + jax.lax.broadcasted_iota(jnp.int32, (bq, bk), 0)
        col = j * bk + jax.lax.broadcasted_iota(jnp.int32, (bq, bk), 1)
        s = jnp.where(col > row, -1e9, s)
        m_prev = m_ref[:, :1]
        m_cur = jnp.max(s, axis=1, keepdims=True)
        m_new = jnp.maximum(m_prev, m_cur)
        p = jnp.exp(s - m_new)
        corr = jnp.exp(m_prev - m_new)
        l_ref[:, :1] = l_ref[:, :1] * corr + jnp.sum(p, axis=1, keepdims=True)
        acc_ref[...] = acc_ref[...] * corr + jax.lax.dot_general(
            p.astype(BF16), v_ref[...], (((1,), (0,)), ((), ())),
            preferred_element_type=F32)
        m_ref[:, :1] = m_new

    @pl.when(j == i)
    def _():
        o_ref[...] = (acc_ref[...] / l_ref[:, :1]).astype(BF16)


def _flash(qkv):
    BQ = 1024
    BK = 1024
    body = functools.partial(_flash_body, bq=BQ, bk=BK)
    return pl.pallas_call(
        body,
        grid=(NH, T // BQ, T // BK),
        in_specs=[
            pl.BlockSpec((BQ, DH), lambda h, i, j: (i, h)),
            pl.BlockSpec((BK, DH), lambda h, i, j: (j, NH + h)),
            pl.BlockSpec((BK, DH), lambda h, i, j: (j, 2 * NH + h)),
        ],
        out_specs=pl.BlockSpec((BQ, DH), lambda h, i, j: (i, h)),
        out_shape=jax.ShapeDtypeStruct((T, H), BF16),
        scratch_shapes=[
            pltpu.VMEM((BQ, 1), F32),
            pltpu.VMEM((BQ, 1), F32),
            pltpu.VMEM((BQ, DH), F32),
        ],
        compiler_params=pltpu.CompilerParams(
            dimension_semantics=("arbitrary", "arbitrary", "arbitrary")),
    )(qkv, qkv, qkv)


def _proj_ln2_body(o_ref, x_ref, wo_ref, g_ref, wr_ref,
                   h1_ref, x2n_ref, lg_ref):
    a = jax.lax.dot_general(o_ref[...], wo_ref[...], (((1,), (0,)), ((), ())),
                            preferred_element_type=F32)
    h1 = x_ref[...] + a
    h1_ref[...] = h1
    v = jnp.mean(h1 * h1, axis=1, keepdims=True)
    xn = h1 * jax.lax.rsqrt(v + EPS) * g_ref[...]
    x2n_ref[...] = xn.astype(BF16)
    lg_ref[...] = jax.lax.dot_general(xn, wr_ref[...], (((1,), (0,)), ((), ())),
                                      preferred_element_type=F32)


def _proj_ln2(o, x, wo, g2, wr):
    BM = 512
    return pl.pallas_call(
        _proj_ln2_body,
        grid=(T // BM,),
        in_specs=[
            pl.BlockSpec((BM, H), lambda i: (i, 0)),
            pl.BlockSpec((BM, H), lambda i: (i, 0)),
            pl.BlockSpec((H, H), lambda i: (0, 0)),
            pl.BlockSpec((1, H), lambda i: (0, 0)),
            pl.BlockSpec((H, E), lambda i: (0, 0)),
        ],
        out_specs=[
            pl.BlockSpec((BM, H), lambda i: (i, 0)),
            pl.BlockSpec((BM, H), lambda i: (i, 0)),
            pl.BlockSpec((BM, E), lambda i: (i, 0)),
        ],
        out_shape=[
            jax.ShapeDtypeStruct((T, H), F32),
            jax.ShapeDtypeStruct((T, H), BF16),
            jax.ShapeDtypeStruct((T, E), F32),
        ],
    )(o, x, wo, g2.reshape(1, H), wr)


def _route_body(lg_ref, comb_ref):
    l = lg_ref[...]
    col = jax.lax.broadcasted_iota(jnp.int32, (T, E), 1)
    m1 = jnp.max(l, axis=1, keepdims=True)
    a1 = jnp.min(jnp.where(l == m1, col, E), axis=1, keepdims=True)
    sel1 = col == a1
    l2 = jnp.where(sel1, -1e30, l)
    m2 = jnp.max(l2, axis=1, keepdims=True)
    a2 = jnp.min(jnp.where(l2 == m2, col, E), axis=1, keepdims=True)
    sel2 = col == a2
    w1 = jax.nn.sigmoid(m1 - m2)
    w2 = 1.0 - w1
    comb_ref[...] = jnp.where(sel1, w1, 0.0) + jnp.where(sel2, w2, 0.0)


def _route(logits):
    return pl.pallas_call(
        _route_body,
        out_shape=jax.ShapeDtypeStruct((T, E), F32),
    )(logits)


def _moe_body(x_ref, w1_ref, w2_ref, comb_ref, h1_ref, o_ref):
    e = pl.program_id(1)
    x = x_ref[...]
    gu = jax.lax.dot_general(x, w1_ref[0], (((1,), (1,)), ((), ())),
                             preferred_element_type=F32)
    g = gu[:, :DFF]
    u = gu[:, DFF:]
    act = (g * jax.nn.sigmoid(g) * u).astype(BF16)
    dn = jax.lax.dot_general(act, w2_ref[0], (((1,), (1,)), ((), ())),
                             preferred_element_type=F32)
    c = comb_ref[...]
    onehot = (jax.lax.broadcasted_iota(jnp.int32, c.shape, 1) == e)
    wgt = jnp.sum(jnp.where(onehot, c, 0.0), axis=1, keepdims=True)
    contrib = dn * wgt

    @pl.when(e == 0)
    def _():
        o_ref[...] = h1_ref[...] + contrib

    @pl.when(e > 0)
    def _():
        o_ref[...] = o_ref[...] + contrib


def _moe(x2n, w1, w2, comb, h1):
    BM = 512
    return pl.pallas_call(
        _moe_body,
        grid=(T // BM, E),
        in_specs=[
            pl.BlockSpec((BM, H), lambda i, e: (i, 0)),
            pl.BlockSpec((1, 2 * DFF, H), lambda i, e: (e, 0, 0)),
            pl.BlockSpec((1, H, DFF), lambda i, e: (e, 0, 0)),
            pl.BlockSpec((BM, E), lambda i, e: (i, 0)),
            pl.BlockSpec((BM, H), lambda i, e: (i, 0)),
        ],
        out_specs=pl.BlockSpec((BM, H), lambda i, e: (i, 0)),
        out_shape=jax.ShapeDtypeStruct((T, H), F32),
        compiler_params=pltpu.CompilerParams(
            dimension_semantics=("arbitrary", "arbitrary")),
    )(x2n, w1, w2, comb, h1)


def kernel(hidden_states, ln1_g, ln2_g, wq, wk, wv, wo, w_router, w1, w2):
    wqkv = jnp.concatenate([wq, wk, wv], axis=1).astype(BF16)
    qkv = _ln_qkv(hidden_states, ln1_g, wqkv)
    o = _flash(qkv)
    h1, x2n, logits = _proj_ln2(o, hidden_states, wo.astype(BF16), ln2_g,
                                w_router)
    comb = _route(logits)
    out = _moe(x2n, w1.astype(BF16), w2.astype(BF16), comb, h1)
    return out
